# baseline (speedup 1.0000x reference)
import jax
import jax.numpy as jnp
from jax.experimental import pallas as pl
from jax.experimental.pallas import tpu as pltpu

_B, _I, _DIM = 1, 2048, 1024
_H, _DH = 8, 64
_NB = 64
_INNER = _H * _DH
_T = 512


def _fused_body(x_ref, wqv_ref, b_ref, k_ref, wo_ref, bout_ref, out_ref,
                wv_bf_ref, wo_bf_ref, diag_ref):
    @pl.when(pl.program_id(0) == 0)
    def _init():
        wv_bf_ref[...] = wqv_ref[:, _INNER:].astype(jnp.bfloat16)
        wo_bf_ref[...] = wo_ref[...].astype(jnp.bfloat16)
        b_full = b_ref[...]
        d_iota = jax.lax.broadcasted_iota(jnp.int32, (1, _DH, _DH), 1)
        e_iota = jax.lax.broadcasted_iota(jnp.int32, (1, _DH, _DH), 2)
        eye = (d_iota == e_iota).astype(jnp.float32)
        diag_ref[...] = jnp.sum(b_full * eye, axis=2)

    xb = x_ref[...]
    q = jax.lax.dot_general(
        xb, wqv_ref[:, :_INNER], (((1,), (0,)), ((), ())),
        preferred_element_type=jnp.float32)
    v = jax.lax.dot_general(
        xb.astype(jnp.bfloat16), wv_bf_ref[...], (((1,), (0,)), ((), ())),
        preferred_element_type=jnp.float32)

    scale = jnp.float32(_DH ** -0.5)
    k_mat = k_ref[...]
    diag_b = diag_ref[...]

    outs = []
    for h in range(_H):
        q_h = q[:, h * _DH:(h + 1) * _DH]
        v_h = v[:, h * _DH:(h + 1) * _DH]
        sim = jax.lax.dot_general(
            q_h, k_mat, (((1,), (1,)), ((), ())),
            preferred_element_type=jnp.float32) * scale
        m1 = jnp.max(sim, axis=1, keepdims=True)
        mask1 = sim == m1
        sim2 = jnp.where(mask1, -jnp.inf, sim)
        m2 = jnp.max(sim2, axis=1, keepdims=True)
        mask2 = sim2 == m2
        u = jnp.exp(sim - m1)
        w = jnp.where(mask1 | mask2, u, 0.0)
        eff = jax.lax.dot_general(
            w, diag_b, (((1,), (0,)), ((), ())),
            preferred_element_type=jnp.float32)
        rnorm = 1.0 / (1.0 + jnp.exp(m2 - m1))
        outs.append((eff * v_h * rnorm).astype(jnp.bfloat16))

    acc = jnp.concatenate(outs, axis=1)
    res = jax.lax.dot_general(
        acc, wo_bf_ref[...], (((1,), (0,)), ((), ())),
        preferred_element_type=jnp.float32)
    out_ref[...] = res + bout_ref[...]


def kernel(x, b, k, W_qv, W_out, b_out):
    x2 = x.reshape(_I, _DIM)
    bout2 = b_out.reshape(1, _DIM)
    grid = (_I // _T,)
    out = pl.pallas_call(
        _fused_body,
        grid=grid,
        in_specs=[
            pl.BlockSpec((_T, _DIM), lambda i: (i, 0)),
            pl.BlockSpec((_DIM, 2 * _INNER), lambda i: (0, 0)),
            pl.BlockSpec((_NB, _DH, _DH), lambda i: (0, 0, 0)),
            pl.BlockSpec((_NB, _DH), lambda i: (0, 0)),
            pl.BlockSpec((_INNER, _DIM), lambda i: (0, 0)),
            pl.BlockSpec((1, _DIM), lambda i: (0, 0)),
        ],
        out_specs=pl.BlockSpec((_T, _DIM), lambda i: (i, 0)),
        out_shape=jax.ShapeDtypeStruct((_I, _DIM), jnp.float32),
        scratch_shapes=[
            pltpu.VMEM((_DIM, _INNER), jnp.bfloat16),
            pltpu.VMEM((_INNER, _DIM), jnp.bfloat16),
            pltpu.VMEM((_NB, _DH), jnp.float32),
        ],
        compiler_params=pltpu.CompilerParams(
            dimension_semantics=("arbitrary",),
        ),
    )(x2, W_qv, b, k, W_out, bout2)
    return out.reshape(_B, _I, _DIM)


# speedup vs baseline: 1.0397x; 1.0397x over previous
import jax
import jax.numpy as jnp
from jax.experimental import pallas as pl
from jax.experimental.pallas import tpu as pltpu

_B, _I, _DIM = 1, 2048, 1024
_H, _DH = 8, 64
_NB = 64
_INNER = _H * _DH
_T = 512


def _fused_body(x_ref, wqv_ref, b_ref, k_ref, wo_ref, bout_ref, out_ref,
                wo_bf_ref, diag_ref):
    @pl.when(pl.program_id(0) == 0)
    def _init():
        wo_bf_ref[...] = wo_ref[...].astype(jnp.bfloat16)
        b_full = b_ref[...]
        d_iota = jax.lax.broadcasted_iota(jnp.int32, (1, _DH, _DH), 1)
        e_iota = jax.lax.broadcasted_iota(jnp.int32, (1, _DH, _DH), 2)
        eye = (d_iota == e_iota).astype(jnp.float32)
        diag_ref[...] = jnp.sum(b_full * eye, axis=2)

    xb = x_ref[...]
    qv = jax.lax.dot_general(
        xb, wqv_ref[...], (((1,), (0,)), ((), ())),
        preferred_element_type=jnp.float32)
    q = qv[:, :_INNER]
    v = qv[:, _INNER:]

    scale = jnp.float32(_DH ** -0.5)
    k_mat = k_ref[...]
    diag_b = diag_ref[...]

    outs = []
    for h in range(_H):
        q_h = q[:, h * _DH:(h + 1) * _DH]
        v_h = v[:, h * _DH:(h + 1) * _DH]
        sim = jax.lax.dot_general(
            q_h, k_mat, (((1,), (1,)), ((), ())),
            preferred_element_type=jnp.float32) * scale
        m1 = jnp.max(sim, axis=1, keepdims=True)
        mask1 = sim == m1
        sim2 = jnp.where(mask1, -jnp.inf, sim)
        m2 = jnp.max(sim2, axis=1, keepdims=True)
        mask2 = sim2 == m2
        e2 = jnp.exp(m2 - m1)
        denom = 1.0 + e2
        a1 = 1.0 / denom
        a2 = e2 / denom
        w = jnp.where(mask1, a1, 0.0) + jnp.where(mask2, a2, 0.0)
        eff = jax.lax.dot_general(
            w, diag_b, (((1,), (0,)), ((), ())),
            preferred_element_type=jnp.float32)
        outs.append((eff * v_h).astype(jnp.bfloat16))

    acc = jnp.concatenate(outs, axis=1)
    res = jax.lax.dot_general(
        acc, wo_bf_ref[...], (((1,), (0,)), ((), ())),
        preferred_element_type=jnp.float32)
    out_ref[...] = res + bout_ref[...]


def kernel(x, b, k, W_qv, W_out, b_out):
    x2 = x.reshape(_I, _DIM)
    bout2 = b_out.reshape(1, _DIM)
    grid = (_I // _T,)
    out = pl.pallas_call(
        _fused_body,
        grid=grid,
        in_specs=[
            pl.BlockSpec((_T, _DIM), lambda i: (i, 0)),
            pl.BlockSpec((_DIM, 2 * _INNER), lambda i: (0, 0)),
            pl.BlockSpec((_NB, _DH, _DH), lambda i: (0, 0, 0)),
            pl.BlockSpec((_NB, _DH), lambda i: (0, 0)),
            pl.BlockSpec((_INNER, _DIM), lambda i: (0, 0)),
            pl.BlockSpec((1, _DIM), lambda i: (0, 0)),
        ],
        out_specs=pl.BlockSpec((_T, _DIM), lambda i: (i, 0)),
        out_shape=jax.ShapeDtypeStruct((_I, _DIM), jnp.float32),
        scratch_shapes=[
            pltpu.VMEM((_INNER, _DIM), jnp.bfloat16),
            pltpu.VMEM((_NB, _DH), jnp.float32),
        ],
        compiler_params=pltpu.CompilerParams(
            dimension_semantics=("arbitrary",),
        ),
    )(x2, W_qv, b, k, W_out, bout2)
    return out.reshape(_B, _I, _DIM)


# speedup vs baseline: 1.0551x; 1.0148x over previous
import jax
import jax.numpy as jnp
from jax.experimental import pallas as pl
from jax.experimental.pallas import tpu as pltpu

_B, _I, _DIM = 1, 2048, 1024
_H, _DH = 8, 64
_NB = 64
_INNER = _H * _DH
_T = 512


def _fused_body(x_ref, wqv_ref, b_ref, k_ref, wo_ref, bout_ref, out_ref,
                wo_bf_ref, diag_ref):
    @pl.when(pl.program_id(0) == 0)
    def _init():
        wo_bf_ref[...] = wo_ref[...].astype(jnp.bfloat16)
        b_full = b_ref[...]
        d_iota = jax.lax.broadcasted_iota(jnp.int32, (1, _DH, _DH), 1)
        e_iota = jax.lax.broadcasted_iota(jnp.int32, (1, _DH, _DH), 2)
        eye = (d_iota == e_iota).astype(jnp.float32)
        diag_ref[...] = jnp.sum(b_full * eye, axis=2)

    xb = x_ref[...]
    qv = jax.lax.dot_general(
        xb, wqv_ref[...], (((1,), (0,)), ((), ())),
        preferred_element_type=jnp.float32)
    q = qv[:, :_INNER]
    v = qv[:, _INNER:]

    k_mat = k_ref[...] * jnp.float32(_DH ** -0.5)
    diag_b = diag_ref[...]

    outs = []
    for h in range(_H):
        q_h = q[:, h * _DH:(h + 1) * _DH]
        v_h = v[:, h * _DH:(h + 1) * _DH]
        sim = jax.lax.dot_general(
            q_h, k_mat, (((1,), (1,)), ((), ())),
            preferred_element_type=jnp.float32)
        m1 = jnp.max(sim, axis=1, keepdims=True)
        mask1 = sim == m1
        sim2 = jnp.where(mask1, -jnp.inf, sim)
        m2 = jnp.max(sim2, axis=1, keepdims=True)
        mask2 = sim2 == m2
        e2 = jnp.exp(m2 - m1)
        denom = 1.0 + e2
        a1 = 1.0 / denom
        a2 = e2 / denom
        w = jnp.where(mask1, a1, 0.0) + jnp.where(mask2, a2, 0.0)
        eff = jax.lax.dot_general(
            w, diag_b, (((1,), (0,)), ((), ())),
            preferred_element_type=jnp.float32)
        outs.append((eff * v_h).astype(jnp.bfloat16))

    acc = jnp.concatenate(outs, axis=1)
    res = jax.lax.dot_general(
        acc, wo_bf_ref[...], (((1,), (0,)), ((), ())),
        preferred_element_type=jnp.float32)
    out_ref[...] = res + bout_ref[...]


def kernel(x, b, k, W_qv, W_out, b_out):
    x2 = x.reshape(_I, _DIM)
    bout2 = b_out.reshape(1, _DIM)
    grid = (_I // _T,)
    out = pl.pallas_call(
        _fused_body,
        grid=grid,
        in_specs=[
            pl.BlockSpec((_T, _DIM), lambda i: (i, 0)),
            pl.BlockSpec((_DIM, 2 * _INNER), lambda i: (0, 0)),
            pl.BlockSpec((_NB, _DH, _DH), lambda i: (0, 0, 0)),
            pl.BlockSpec((_NB, _DH), lambda i: (0, 0)),
            pl.BlockSpec((_INNER, _DIM), lambda i: (0, 0)),
            pl.BlockSpec((1, _DIM), lambda i: (0, 0)),
        ],
        out_specs=pl.BlockSpec((_T, _DIM), lambda i: (i, 0)),
        out_shape=jax.ShapeDtypeStruct((_I, _DIM), jnp.float32),
        scratch_shapes=[
            pltpu.VMEM((_INNER, _DIM), jnp.bfloat16),
            pltpu.VMEM((_NB, _DH), jnp.float32),
        ],
        compiler_params=pltpu.CompilerParams(
            dimension_semantics=("arbitrary",),
        ),
    )(x2, W_qv, b, k, W_out, bout2)
    return out.reshape(_B, _I, _DIM)


# speedup vs baseline: 1.0759x; 1.0197x over previous
import jax
import jax.numpy as jnp
from jax.experimental import pallas as pl
from jax.experimental.pallas import tpu as pltpu

_B, _I, _DIM = 1, 2048, 1024
_H, _DH = 8, 64
_NB = 64
_INNER = _H * _DH
_T = 512


def _fused_body(x_ref, wqv_ref, b_ref, k_ref, wo_ref, bout_ref, out_ref,
                wo_bf_ref, diag_t_ref):
    @pl.when(pl.program_id(0) == 0)
    def _init():
        wo_bf_ref[...] = wo_ref[...].astype(jnp.bfloat16)
        b_full = b_ref[...]
        d_iota = jax.lax.broadcasted_iota(jnp.int32, (1, _DH, _DH), 1)
        e_iota = jax.lax.broadcasted_iota(jnp.int32, (1, _DH, _DH), 2)
        eye = (d_iota == e_iota).astype(jnp.float32)
        diag_b = jnp.sum(b_full * eye, axis=2)
        diag_t_ref[...] = jnp.transpose(diag_b)

    xb = x_ref[...]
    qv = jax.lax.dot_general(
        xb, wqv_ref[...], (((1,), (0,)), ((), ())),
        preferred_element_type=jnp.float32)
    q_t = jnp.transpose(qv[:, :_INNER])
    v = qv[:, _INNER:]

    k_s = k_ref[...] * jnp.float32(_DH ** -0.5)
    diag_t = diag_t_ref[...]

    effs = []
    for h in range(_H):
        sim = jax.lax.dot_general(
            k_s, q_t[h * _DH:(h + 1) * _DH, :], (((1,), (0,)), ((), ())),
            preferred_element_type=jnp.float32)
        m1 = jnp.max(sim, axis=0, keepdims=True)
        mask1 = sim == m1
        sim2 = jnp.where(mask1, -jnp.inf, sim)
        m2 = jnp.max(sim2, axis=0, keepdims=True)
        mask2 = sim2 == m2
        e2 = jnp.exp(m2 - m1)
        denom = 1.0 + e2
        a1 = 1.0 / denom
        a2 = e2 / denom
        w = jnp.where(mask1, a1, 0.0) + jnp.where(mask2, a2, 0.0)
        effs.append(jax.lax.dot_general(
            diag_t, w, (((1,), (0,)), ((), ())),
            preferred_element_type=jnp.float32))

    eff_t = jnp.concatenate(effs, axis=0)
    eff = jnp.transpose(eff_t)
    acc = (eff * v).astype(jnp.bfloat16)
    res = jax.lax.dot_general(
        acc, wo_bf_ref[...], (((1,), (0,)), ((), ())),
        preferred_element_type=jnp.float32)
    out_ref[...] = res + bout_ref[...]


def kernel(x, b, k, W_qv, W_out, b_out):
    x2 = x.reshape(_I, _DIM)
    bout2 = b_out.reshape(1, _DIM)
    grid = (_I // _T,)
    out = pl.pallas_call(
        _fused_body,
        grid=grid,
        in_specs=[
            pl.BlockSpec((_T, _DIM), lambda i: (i, 0)),
            pl.BlockSpec((_DIM, 2 * _INNER), lambda i: (0, 0)),
            pl.BlockSpec((_NB, _DH, _DH), lambda i: (0, 0, 0)),
            pl.BlockSpec((_NB, _DH), lambda i: (0, 0)),
            pl.BlockSpec((_INNER, _DIM), lambda i: (0, 0)),
            pl.BlockSpec((1, _DIM), lambda i: (0, 0)),
        ],
        out_specs=pl.BlockSpec((_T, _DIM), lambda i: (i, 0)),
        out_shape=jax.ShapeDtypeStruct((_I, _DIM), jnp.float32),
        scratch_shapes=[
            pltpu.VMEM((_INNER, _DIM), jnp.bfloat16),
            pltpu.VMEM((_DH, _NB), jnp.float32),
        ],
        compiler_params=pltpu.CompilerParams(
            dimension_semantics=("arbitrary",),
        ),
    )(x2, W_qv, b, k, W_out, bout2)
    return out.reshape(_B, _I, _DIM)
